# Initial kernel scaffold; baseline (speedup 1.0000x reference)
import jax
import jax.numpy as jnp
from jax.experimental import pallas as pl


def kernel(input, wg):
    raise NotImplementedError("write your pallas kernel here")


# speedup vs baseline: 2.0148x; 2.0148x over previous
import functools

import jax
import jax.numpy as jnp
from jax import lax
from jax.experimental import pallas as pl

TOKENS = 32768
D_MODEL = 768
NUM_EXPERTS = 64
BLOCK = 2048


def _gate_body(x_ref, w_ref, i1_ref, i2_ref, g1_ref, g2_ref):
    x_blk = x_ref[...]
    w = w_ref[...]
    logits = jnp.dot(x_blk, w, preferred_element_type=jnp.float32)
    b, e = logits.shape
    iota = lax.broadcasted_iota(jnp.int32, (b, e), 1)
    m1 = jnp.max(logits, axis=1, keepdims=True)
    i1 = jnp.min(jnp.where(logits == m1, iota, e), axis=1)
    masked = jnp.where(iota == i1[:, None], -jnp.inf, logits)
    m2 = jnp.max(masked, axis=1, keepdims=True)
    i2 = jnp.min(jnp.where(masked == m2, iota, e), axis=1)
    ex = jnp.exp(m2[:, 0] - m1[:, 0])
    g1 = 1.0 / (1.0 + ex)
    i1_ref[...] = i1.astype(jnp.int32)
    i2_ref[...] = i2.astype(jnp.int32)
    g1_ref[...] = g1
    g2_ref[...] = 1.0 - g1


@functools.partial(jax.jit, static_argnames=("interpret",))
def kernel(input, wg, interpret=False):
    n, d = input.shape
    e = wg.shape[0]
    wg_t = wg.T
    grid = (n // BLOCK,)
    out_shapes = (
        jax.ShapeDtypeStruct((n,), jnp.int32),
        jax.ShapeDtypeStruct((n,), jnp.int32),
        jax.ShapeDtypeStruct((n,), jnp.float32),
        jax.ShapeDtypeStruct((n,), jnp.float32),
    )
    vec_spec = pl.BlockSpec((BLOCK,), lambda i: (i,))
    return pl.pallas_call(
        _gate_body,
        grid=grid,
        in_specs=[
            pl.BlockSpec((BLOCK, d), lambda i: (i, 0)),
            pl.BlockSpec((d, e), lambda i: (0, 0)),
        ],
        out_specs=(vec_spec, vec_spec, vec_spec, vec_spec),
        out_shape=out_shapes,
        interpret=interpret,
    )(input, wg_t)


# speedup vs baseline: 4.9352x; 2.4495x over previous
import functools

import jax
import jax.numpy as jnp
from jax import lax
from jax.experimental import pallas as pl

TOKENS = 32768
D_MODEL = 768
NUM_EXPERTS = 64
BLOCK = 2048


def _gate_body(x_ref, w_ref, i1_ref, i2_ref, g1_ref, g2_ref):
    x_blk = x_ref[...]
    w = w_ref[...]
    logits = lax.dot_general(w, x_blk, (((1,), (1,)), ((), ())),
                             preferred_element_type=jnp.float32)
    e, b = logits.shape
    iota = lax.broadcasted_iota(jnp.int32, (e, b), 0)
    m1 = jnp.max(logits, axis=0)
    i1 = jnp.min(jnp.where(logits == m1[None, :], iota, e), axis=0)
    masked = jnp.where(iota == i1[None, :], -jnp.inf, logits)
    m2 = jnp.max(masked, axis=0)
    i2 = jnp.min(jnp.where(masked == m2[None, :], iota, e), axis=0)
    ex = jnp.exp(m2 - m1)
    g1 = 1.0 / (1.0 + ex)
    i1_ref[...] = i1.astype(jnp.int32)
    i2_ref[...] = i2.astype(jnp.int32)
    g1_ref[...] = g1
    g2_ref[...] = 1.0 - g1


@functools.partial(jax.jit, static_argnames=("interpret",))
def kernel(input, wg, interpret=False):
    n, d = input.shape
    e = wg.shape[0]
    grid = (n // BLOCK,)
    out_shapes = (
        jax.ShapeDtypeStruct((n,), jnp.int32),
        jax.ShapeDtypeStruct((n,), jnp.int32),
        jax.ShapeDtypeStruct((n,), jnp.float32),
        jax.ShapeDtypeStruct((n,), jnp.float32),
    )
    vec_spec = pl.BlockSpec((BLOCK,), lambda i: (i,))
    return pl.pallas_call(
        _gate_body,
        grid=grid,
        in_specs=[
            pl.BlockSpec((BLOCK, d), lambda i: (i, 0)),
            pl.BlockSpec((e, d), lambda i: (0, 0)),
        ],
        out_specs=(vec_spec, vec_spec, vec_spec, vec_spec),
        out_shape=out_shapes,
        interpret=interpret,
    )(input, wg)


# speedup vs baseline: 5.2003x; 1.0537x over previous
import functools

import jax
import jax.numpy as jnp
from jax import lax
from jax.experimental import pallas as pl

TOKENS = 32768
D_MODEL = 768
NUM_EXPERTS = 64
BLOCK = 4096


def _gate_body(x_ref, w_ref, i1_ref, i2_ref, g1_ref, g2_ref):
    x_blk = x_ref[...]
    w = w_ref[...]
    logits = lax.dot_general(w, x_blk, (((1,), (1,)), ((), ())),
                             preferred_element_type=jnp.float32)
    e, b = logits.shape
    iota = lax.broadcasted_iota(jnp.int32, (e, b), 0)
    m1 = jnp.max(logits, axis=0)
    i1 = jnp.min(jnp.where(logits == m1[None, :], iota, e), axis=0)
    masked = jnp.where(iota == i1[None, :], -jnp.inf, logits)
    m2 = jnp.max(masked, axis=0)
    i2 = jnp.min(jnp.where(masked == m2[None, :], iota, e), axis=0)
    ex = jnp.exp(m2 - m1)
    g1 = 1.0 / (1.0 + ex)
    i1_ref[...] = i1.astype(jnp.int32)
    i2_ref[...] = i2.astype(jnp.int32)
    g1_ref[...] = g1
    g2_ref[...] = 1.0 - g1


@functools.partial(jax.jit, static_argnames=("interpret",))
def kernel(input, wg, interpret=False):
    n, d = input.shape
    e = wg.shape[0]
    grid = (n // BLOCK,)
    out_shapes = (
        jax.ShapeDtypeStruct((n,), jnp.int32),
        jax.ShapeDtypeStruct((n,), jnp.int32),
        jax.ShapeDtypeStruct((n,), jnp.float32),
        jax.ShapeDtypeStruct((n,), jnp.float32),
    )
    vec_spec = pl.BlockSpec((BLOCK,), lambda i: (i,))
    return pl.pallas_call(
        _gate_body,
        grid=grid,
        in_specs=[
            pl.BlockSpec((BLOCK, d), lambda i: (i, 0)),
            pl.BlockSpec((e, d), lambda i: (0, 0)),
        ],
        out_specs=(vec_spec, vec_spec, vec_spec, vec_spec),
        out_shape=out_shapes,
        interpret=interpret,
    )(input, wg)
